# baseline (speedup 1.0000x reference)
import dataclasses
import functools

import jax
import jax.numpy as jnp
from jax import lax
from jax.experimental import pallas as pl
from jax.experimental.pallas import tpu as pltpu
from jax.experimental.pallas import tpu_sc as plsc

N = 10000
F = 128
NC = 2
NS = 16
LANES = 16
CHUNK = 128
CHUNKS_PER_TILE = 80
E_PAD = NC * NS * CHUNKS_PER_TILE * CHUNK
ROWS_PER_TILE_AGG = 632
N_ACC = NS * ROWS_PER_TILE_AGG
NB = 10016
EPT = E_PAD // (NC * NS)

_mesh = plsc.VectorSubcoreMesh(core_axis_name="c", subcore_axis_name="s")

_cp = pltpu.CompilerParams()
if "needs_layout_passes" in pltpu.CompilerParams.__dataclass_fields__:
    _cp = dataclasses.replace(_cp, needs_layout_passes=False)


@functools.partial(
    pl.kernel,
    out_type=jax.ShapeDtypeStruct((NC, NS, 2, NB), jnp.float32),
    mesh=_mesh,
    compiler_params=_cp,
    scratch_types=[
        pltpu.VMEM((EPT,), jnp.int32),
        pltpu.VMEM((EPT,), jnp.int32),
        pltpu.VMEM((2, NB), jnp.float32),
    ],
)
def _count_kernel(src_hbm, dst_hbm, zeros_hbm, out_hbm, si_v, di_v, hist_v):
    c = lax.axis_index("c")
    s = lax.axis_index("s")
    pltpu.sync_copy(zeros_hbm, hist_v)
    pltpu.sync_copy(src_hbm.at[c].at[s], si_v)
    pltpu.sync_copy(dst_hbm.at[c].at[s], di_v)
    ones16 = jnp.full((16,), 1.0, jnp.float32)
    row0 = jnp.zeros((16,), jnp.int32)
    row1 = jnp.ones((16,), jnp.int32)

    @pl.loop(0, EPT, step=16)
    def _(e):
        sv = si_v[pl.ds(e, 16)]
        dv = di_v[pl.ds(e, 16)]
        plsc.addupdate_scatter(hist_v, [row0, sv], ones16)
        plsc.addupdate_scatter(hist_v, [row1, dv], ones16)

    pltpu.sync_copy(hist_v, out_hbm.at[c].at[s])


HALF = CHUNKS_PER_TILE // 2


def _agg_body(h_hbm, si_v, di_v, rows0_v, rows1_v, acc_sh,
              g0, g1, s0, s1):
    pltpu.make_async_copy(h_hbm.at[si_v.at[0]], rows0_v, g0).start()
    pltpu.make_async_copy(h_hbm.at[si_v.at[1]], rows1_v, g1).start()

    @pl.loop(0, HALF - 2, step=2)
    def _(j):
        pltpu.make_async_copy(h_hbm.at[si_v.at[j]], rows0_v, g0).wait()
        sc0 = pltpu.async_copy(rows0_v, acc_sh.at[di_v.at[j]], s0, add=True)
        pltpu.make_async_copy(h_hbm.at[si_v.at[j + 1]], rows1_v, g1).wait()
        sc1 = pltpu.async_copy(rows1_v, acc_sh.at[di_v.at[j + 1]], s1,
                               add=True)
        sc0.wait()
        pltpu.make_async_copy(h_hbm.at[si_v.at[j + 2]], rows0_v, g0).start()
        sc1.wait()
        pltpu.make_async_copy(h_hbm.at[si_v.at[j + 3]], rows1_v, g1).start()

    pltpu.make_async_copy(h_hbm.at[si_v.at[HALF - 2]], rows0_v, g0).wait()
    pltpu.sync_copy(rows0_v, acc_sh.at[di_v.at[HALF - 2]], add=True)
    pltpu.make_async_copy(h_hbm.at[si_v.at[HALF - 1]], rows1_v, g1).wait()
    pltpu.sync_copy(rows1_v, acc_sh.at[di_v.at[HALF - 1]], add=True)


@functools.partial(
    pl.kernel,
    out_type=jax.ShapeDtypeStruct((NC, N_ACC, F), jnp.float32),
    mesh=_mesh,
    scratch_types=[
        pltpu.VMEM((HALF, CHUNK), jnp.int32),
        pltpu.VMEM((HALF, CHUNK), jnp.int32),
        pltpu.VMEM((CHUNK, F), jnp.float32),
        pltpu.VMEM((CHUNK, F), jnp.float32),
        pltpu.VMEM_SHARED((N_ACC, F), jnp.float32),
        pltpu.SemaphoreType.DMA,
        pltpu.SemaphoreType.DMA,
        pltpu.SemaphoreType.DMA,
        pltpu.SemaphoreType.DMA,
    ],
)
def _agg_kernel(h_hbm, src_hbm, dst_hbm, zeros_hbm, out_hbm,
                si_v, di_v, rows0_v, rows1_v, acc_sh, g0, g1, s0, s1):
    c = lax.axis_index("c")
    s = lax.axis_index("s")
    base = s * ROWS_PER_TILE_AGG

    pltpu.sync_copy(zeros_hbm, acc_sh.at[pl.ds(base, ROWS_PER_TILE_AGG)])
    pltpu.sync_copy(src_hbm.at[c].at[s].at[pl.ds(0, HALF)], si_v)
    pltpu.sync_copy(dst_hbm.at[c].at[s].at[pl.ds(0, HALF)], di_v)
    plsc.subcore_barrier()

    _agg_body(h_hbm, si_v, di_v, rows0_v, rows1_v, acc_sh, g0, g1, s0, s1)

    pltpu.sync_copy(src_hbm.at[c].at[s].at[pl.ds(HALF, HALF)], si_v)
    pltpu.sync_copy(dst_hbm.at[c].at[s].at[pl.ds(HALF, HALF)], di_v)

    _agg_body(h_hbm, si_v, di_v, rows0_v, rows1_v, acc_sh, g0, g1, s0, s1)

    plsc.subcore_barrier()
    pltpu.sync_copy(acc_sh.at[pl.ds(base, ROWS_PER_TILE_AGG)],
                    out_hbm.at[c].at[pl.ds(base, ROWS_PER_TILE_AGG)])


def _norm_from(cnt_ref):
    cnt = jnp.sum(cnt_ref[...], axis=1, keepdims=True)
    return lax.rsqrt(jnp.maximum(cnt, 1.0))


def _mm_scale_body(x_ref, w_ref, cs_ref, o_ref):
    o_ref[...] = jnp.dot(x_ref[...], w_ref[...],
                         preferred_element_type=jnp.float32) * _norm_from(cs_ref)


def _mid_body(p_ref, cd_ref, b_ref, w_ref, cs_ref, o_ref):
    x1 = jnp.maximum((p_ref[0] + p_ref[1]) * _norm_from(cd_ref) + b_ref[...],
                     0.0)
    o_ref[...] = jnp.dot(x1, w_ref[...],
                         preferred_element_type=jnp.float32) * _norm_from(cs_ref)


def _final_body(q_ref, cd_ref, b_ref, o_ref):
    o_ref[...] = jnp.maximum(
        (q_ref[0] + q_ref[1]) * _norm_from(cd_ref) + b_ref[...], 0.0)


_ROWS_BLK = 1000
_GRID = N // _ROWS_BLK

_cnt_spec = pl.BlockSpec((_ROWS_BLK, NC * NS), lambda i: (i, 0))
_row_spec = pl.BlockSpec((_ROWS_BLK, F), lambda i: (i, 0))
_w_spec = pl.BlockSpec((F, F), lambda i: (0, 0))
_b_spec = pl.BlockSpec((1, F), lambda i: (0, 0))
_p_spec = pl.BlockSpec((NC, _ROWS_BLK, F), lambda i: (0, i, 0))
_out_sds = jax.ShapeDtypeStruct((N, F), jnp.float32)

_mm_scale = pl.pallas_call(
    _mm_scale_body,
    grid=(_GRID,),
    in_specs=[_row_spec, _w_spec, _cnt_spec],
    out_specs=_row_spec,
    out_shape=_out_sds,
)

_mid = pl.pallas_call(
    _mid_body,
    grid=(_GRID,),
    in_specs=[_p_spec, _cnt_spec, _b_spec, _w_spec, _cnt_spec],
    out_specs=_row_spec,
    out_shape=_out_sds,
)

_final = pl.pallas_call(
    _final_body,
    grid=(_GRID,),
    in_specs=[_p_spec, _cnt_spec, _b_spec],
    out_specs=_row_spec,
    out_shape=_out_sds,
)


def kernel(features, edge_index, W0, b0, W1, b1):
    src = edge_index[0].astype(jnp.int32)
    dst = edge_index[1].astype(jnp.int32)
    e = src.shape[0]
    npad = E_PAD - e
    pad_iota = lax.iota(jnp.int32, npad)
    idx_shape = (NC, NS, CHUNKS_PER_TILE, CHUNK)
    src_cnt = jnp.concatenate([src, N + (pad_iota % 16)]).reshape(idx_shape)
    src_gat = jnp.concatenate([src, pad_iota % N]).reshape(idx_shape)
    dst_all = jnp.concatenate([dst, N + (pad_iota % 16)]).reshape(idx_shape)

    zeros_cnt = jnp.zeros((2, NB), jnp.float32)
    cnt = _count_kernel(src_cnt.reshape(NC, NS, EPT),
                        dst_all.reshape(NC, NS, EPT),
                        zeros_cnt)
    cnt = jnp.transpose(cnt.reshape(NC * NS, 2, NB), (2, 1, 0))
    cs_all = cnt[:N, 0]
    cd_all = cnt[:N, 1]

    zeros_agg = jnp.zeros((ROWS_PER_TILE_AGG, F), jnp.float32)
    b0r = b0.reshape(1, F)
    b1r = b1.reshape(1, F)

    h0 = _mm_scale(features, W0, cs_all)
    p = _agg_kernel(h0, src_gat, dst_all, zeros_agg)
    h1 = _mid(p, cd_all, b0r, W1, cs_all)
    q = _agg_kernel(h1, src_gat, dst_all, zeros_agg)
    return _final(q, cd_all, b1r)


# speedup vs baseline: 1.2246x; 1.2246x over previous
import dataclasses
import functools

import jax
import jax.numpy as jnp
from jax import lax
from jax.experimental import pallas as pl
from jax.experimental.pallas import tpu as pltpu
from jax.experimental.pallas import tpu_sc as plsc

N = 10000
F = 128
NC = 2
NS = 16
LANES = 16
CHUNK = 128
CHUNKS_PER_TILE = 80
E_PAD = NC * NS * CHUNKS_PER_TILE * CHUNK
ROWS_PER_TILE_AGG = 632
N_ACC = NS * ROWS_PER_TILE_AGG
NB = 10016
EPT = E_PAD // (NC * NS)

_mesh = plsc.VectorSubcoreMesh(core_axis_name="c", subcore_axis_name="s")

_cp = pltpu.CompilerParams()
if "needs_layout_passes" in pltpu.CompilerParams.__dataclass_fields__:
    _cp = dataclasses.replace(_cp, needs_layout_passes=False)


@functools.partial(
    pl.kernel,
    out_type=jax.ShapeDtypeStruct((NC, NS, 2, NB), jnp.float32),
    mesh=_mesh,
    compiler_params=_cp,
    scratch_types=[
        pltpu.VMEM((EPT,), jnp.int32),
        pltpu.VMEM((EPT,), jnp.int32),
        pltpu.VMEM((2, NB), jnp.float32),
    ],
)
def _count_kernel(src_hbm, dst_hbm, zeros_hbm, out_hbm, si_v, di_v, hist_v):
    c = lax.axis_index("c")
    s = lax.axis_index("s")
    pltpu.sync_copy(zeros_hbm, hist_v)
    pltpu.sync_copy(src_hbm.at[c].at[s], si_v)
    pltpu.sync_copy(dst_hbm.at[c].at[s], di_v)
    ones16 = jnp.full((16,), 1.0, jnp.float32)
    row0 = jnp.zeros((16,), jnp.int32)
    row1 = jnp.ones((16,), jnp.int32)

    @pl.loop(0, EPT, step=16)
    def _(e):
        sv = si_v[pl.ds(e, 16)]
        dv = di_v[pl.ds(e, 16)]
        plsc.addupdate_scatter(hist_v, [row0, sv], ones16)
        plsc.addupdate_scatter(hist_v, [row1, dv], ones16)

    pltpu.sync_copy(hist_v, out_hbm.at[c].at[s])


HALF = CHUNKS_PER_TILE // 2


def _agg_body(h_hbm, si_v, di_v, rows0_v, rows1_v, acc_sh, sem0, sem1):
    pltpu.make_async_copy(h_hbm.at[si_v.at[0]], rows0_v, sem0).start()

    @pl.loop(0, HALF, step=2)
    def _(j):
        pltpu.make_async_copy(h_hbm.at[si_v.at[j + 1]], rows1_v, sem1).start()
        pltpu.make_async_copy(h_hbm.at[si_v.at[j]], rows0_v, sem0).wait()
        pltpu.sync_copy(rows0_v, acc_sh.at[di_v.at[j]], add=True)

        @pl.when(j + 2 < HALF)
        def _():
            pltpu.make_async_copy(h_hbm.at[si_v.at[j + 2]], rows0_v,
                                  sem0).start()

        pltpu.make_async_copy(h_hbm.at[si_v.at[j + 1]], rows1_v, sem1).wait()
        pltpu.sync_copy(rows1_v, acc_sh.at[di_v.at[j + 1]], add=True)


@functools.partial(
    pl.kernel,
    out_type=jax.ShapeDtypeStruct((NC, N_ACC, F), jnp.float32),
    mesh=_mesh,
    scratch_types=[
        pltpu.VMEM((HALF, CHUNK), jnp.int32),
        pltpu.VMEM((HALF, CHUNK), jnp.int32),
        pltpu.VMEM((CHUNK, F), jnp.float32),
        pltpu.VMEM((CHUNK, F), jnp.float32),
        pltpu.VMEM_SHARED((N_ACC, F), jnp.float32),
        pltpu.SemaphoreType.DMA,
        pltpu.SemaphoreType.DMA,
    ],
)
def _agg_kernel(h_hbm, src_hbm, dst_hbm, zeros_hbm, out_hbm,
                si_v, di_v, rows0_v, rows1_v, acc_sh, sem0, sem1):
    c = lax.axis_index("c")
    s = lax.axis_index("s")
    base = s * ROWS_PER_TILE_AGG

    pltpu.sync_copy(zeros_hbm, acc_sh.at[pl.ds(base, ROWS_PER_TILE_AGG)])
    pltpu.sync_copy(src_hbm.at[c].at[s].at[pl.ds(0, HALF)], si_v)
    pltpu.sync_copy(dst_hbm.at[c].at[s].at[pl.ds(0, HALF)], di_v)
    plsc.subcore_barrier()

    _agg_body(h_hbm, si_v, di_v, rows0_v, rows1_v, acc_sh, sem0, sem1)

    pltpu.sync_copy(src_hbm.at[c].at[s].at[pl.ds(HALF, HALF)], si_v)
    pltpu.sync_copy(dst_hbm.at[c].at[s].at[pl.ds(HALF, HALF)], di_v)

    _agg_body(h_hbm, si_v, di_v, rows0_v, rows1_v, acc_sh, sem0, sem1)

    plsc.subcore_barrier()
    pltpu.sync_copy(acc_sh.at[pl.ds(base, ROWS_PER_TILE_AGG)],
                    out_hbm.at[c].at[pl.ds(base, ROWS_PER_TILE_AGG)])


def _norm_from(cnt_ref):
    cnt = jnp.sum(cnt_ref[...], axis=1, keepdims=True)
    return lax.rsqrt(jnp.maximum(cnt, 1.0))


def _mm_scale_body(x_ref, w_ref, cs_ref, o_ref):
    o_ref[...] = jnp.dot(x_ref[...], w_ref[...],
                         preferred_element_type=jnp.float32) * _norm_from(cs_ref)


def _mid_body(p_ref, cd_ref, b_ref, w_ref, cs_ref, o_ref):
    x1 = jnp.maximum((p_ref[0] + p_ref[1]) * _norm_from(cd_ref) + b_ref[...],
                     0.0)
    o_ref[...] = jnp.dot(x1, w_ref[...],
                         preferred_element_type=jnp.float32) * _norm_from(cs_ref)


def _final_body(q_ref, cd_ref, b_ref, o_ref):
    o_ref[...] = jnp.maximum(
        (q_ref[0] + q_ref[1]) * _norm_from(cd_ref) + b_ref[...], 0.0)


_ROWS_BLK = 1000
_GRID = N // _ROWS_BLK

_cnt_spec = pl.BlockSpec((_ROWS_BLK, NC * NS), lambda i: (i, 0))
_row_spec = pl.BlockSpec((_ROWS_BLK, F), lambda i: (i, 0))
_w_spec = pl.BlockSpec((F, F), lambda i: (0, 0))
_b_spec = pl.BlockSpec((1, F), lambda i: (0, 0))
_p_spec = pl.BlockSpec((NC, _ROWS_BLK, F), lambda i: (0, i, 0))
_out_sds = jax.ShapeDtypeStruct((N, F), jnp.float32)

_mm_scale = pl.pallas_call(
    _mm_scale_body,
    grid=(_GRID,),
    in_specs=[_row_spec, _w_spec, _cnt_spec],
    out_specs=_row_spec,
    out_shape=_out_sds,
)

_mid = pl.pallas_call(
    _mid_body,
    grid=(_GRID,),
    in_specs=[_p_spec, _cnt_spec, _b_spec, _w_spec, _cnt_spec],
    out_specs=_row_spec,
    out_shape=_out_sds,
)

_final = pl.pallas_call(
    _final_body,
    grid=(_GRID,),
    in_specs=[_p_spec, _cnt_spec, _b_spec],
    out_specs=_row_spec,
    out_shape=_out_sds,
)


def kernel(features, edge_index, W0, b0, W1, b1):
    src = edge_index[0].astype(jnp.int32)
    dst = edge_index[1].astype(jnp.int32)
    e = src.shape[0]
    npad = E_PAD - e
    pad_iota = lax.iota(jnp.int32, npad)
    idx_shape = (NC, NS, CHUNKS_PER_TILE, CHUNK)
    src_cnt = jnp.concatenate([src, N + (pad_iota % 16)]).reshape(idx_shape)
    src_gat = jnp.concatenate([src, pad_iota % N]).reshape(idx_shape)
    dst_all = jnp.concatenate([dst, N + (pad_iota % 16)]).reshape(idx_shape)

    zeros_cnt = jnp.zeros((2, NB), jnp.float32)
    cnt = _count_kernel(src_cnt.reshape(NC, NS, EPT),
                        dst_all.reshape(NC, NS, EPT),
                        zeros_cnt)
    cnt = jnp.transpose(cnt.reshape(NC * NS, 2, NB), (2, 1, 0))
    cs_all = cnt[:N, 0]
    cd_all = cnt[:N, 1]

    zeros_agg = jnp.zeros((ROWS_PER_TILE_AGG, F), jnp.float32)
    b0r = b0.reshape(1, F)
    b1r = b1.reshape(1, F)

    h0 = _mm_scale(features, W0, cs_all)
    p = _agg_kernel(h0, src_gat, dst_all, zeros_agg)
    h1 = _mid(p, cd_all, b0r, W1, cs_all)
    q = _agg_kernel(h1, src_gat, dst_all, zeros_agg)
    return _final(q, cd_all, b1r)
